# baseline (speedup 1.0000x reference)
import jax
import jax.numpy as jnp
from jax.experimental import pallas as pl

_HID = 4096
_NE = 64
_K = 8
_BT = 512


def _router_block(x_ref, wt_ref, rw_ref, se_ref):
    x = x_ref[...]
    wt = wt_ref[...]
    logits = jax.lax.dot_general(x, wt, (((1,), (1,)), ((), ())), preferred_element_type=jnp.float32)
    cur = logits.T
    row = jax.lax.broadcasted_iota(jnp.int32, cur.shape, 0)
    vals = []
    idxs = []
    for _ in range(_K):
        m = jnp.max(cur, axis=0, keepdims=True)
        idx = jnp.min(jnp.where(cur == m, row, _NE), axis=0, keepdims=True)
        vals.append(m)
        idxs.append(idx)
        cur = jnp.where(row == idx, -jnp.inf, cur)
    v = jnp.concatenate(vals, axis=0)
    i = jnp.concatenate(idxs, axis=0)
    e = jnp.exp(v - v[:1])
    w = e / jnp.sum(e, axis=0, keepdims=True)
    rw_ref[...] = w.T
    se_ref[...] = i.T


def kernel(hidden_states, gate_w):
    flat = hidden_states.reshape(-1, _HID)
    n_tok = flat.shape[0]
    wt = gate_w
    rw, se = pl.pallas_call(
        _router_block,
        grid=(n_tok // _BT,),
        in_specs=[
            pl.BlockSpec((_BT, _HID), lambda i: (i, 0)),
            pl.BlockSpec((_NE, _HID), lambda i: (0, 0)),
        ],
        out_specs=[
            pl.BlockSpec((_BT, _K), lambda i: (i, 0)),
            pl.BlockSpec((_BT, _K), lambda i: (i, 0)),
        ],
        out_shape=[
            jax.ShapeDtypeStruct((n_tok, _K), jnp.float32),
            jax.ShapeDtypeStruct((n_tok, _K), jnp.int32),
        ],
    )(flat, wt)
    return (rw, se)


# speedup vs baseline: 1.0458x; 1.0458x over previous
import jax
import jax.numpy as jnp
from jax.experimental import pallas as pl

_HID = 4096
_NE = 64
_K = 8
_BT = 1024


def _router_block(x_ref, wt_ref, rw_ref, se_ref):
    x = x_ref[...]
    wt = wt_ref[...]
    logits = jax.lax.dot_general(x, wt, (((1,), (1,)), ((), ())), preferred_element_type=jnp.float32)
    cur = logits.T
    row = jax.lax.broadcasted_iota(jnp.int32, cur.shape, 0)
    vals = []
    idxs = []
    for _ in range(_K):
        m = jnp.max(cur, axis=0, keepdims=True)
        idx = jnp.min(jnp.where(cur == m, row, _NE), axis=0, keepdims=True)
        vals.append(m)
        idxs.append(idx)
        cur = jnp.where(row == idx, -jnp.inf, cur)
    v = jnp.concatenate(vals, axis=0)
    i = jnp.concatenate(idxs, axis=0)
    e = jnp.exp(v - v[:1])
    w = e / jnp.sum(e, axis=0, keepdims=True)
    rw_ref[...] = w.T
    se_ref[...] = i.T


def kernel(hidden_states, gate_w):
    flat = hidden_states.reshape(-1, _HID)
    n_tok = flat.shape[0]
    wt = gate_w
    rw, se = pl.pallas_call(
        _router_block,
        grid=(n_tok // _BT,),
        in_specs=[
            pl.BlockSpec((_BT, _HID), lambda i: (i, 0)),
            pl.BlockSpec((_NE, _HID), lambda i: (0, 0)),
        ],
        out_specs=[
            pl.BlockSpec((_BT, _K), lambda i: (i, 0)),
            pl.BlockSpec((_BT, _K), lambda i: (i, 0)),
        ],
        out_shape=[
            jax.ShapeDtypeStruct((n_tok, _K), jnp.float32),
            jax.ShapeDtypeStruct((n_tok, _K), jnp.int32),
        ],
    )(flat, wt)
    return (rw, se)
